# baseline (speedup 1.0000x reference)
import jax
import jax.numpy as jnp
from jax.experimental import pallas as pl
from jax.experimental.pallas import tpu as pltpu

_REP = 8


def _pos_kernel(col_ref, row_ref, o_hbm, scratch, sem):
    nc = col_ref.shape[1]
    w = col_ref.shape[0]
    h = row_ref.shape[0]
    rep = scratch.shape[0]
    col_t = col_ref[...].T
    row_t = row_ref[...].T
    scratch[:, :nc] = jnp.broadcast_to(
        col_t[None, :, None, :], (rep, nc, h, w)
    ).reshape(rep, nc, h * w)
    scratch[:, nc:] = jnp.broadcast_to(
        row_t[None, :, :, None], (rep, nc, h, w)
    ).reshape(rep, nc, h * w)
    b_total = o_hbm.shape[0]
    n_copies = b_total // rep
    for i in range(n_copies):
        pltpu.make_async_copy(
            scratch, o_hbm.at[pl.ds(i * rep, rep)], sem
        ).start()
    for i in range(n_copies):
        pltpu.make_async_copy(
            scratch, o_hbm.at[pl.ds(i * rep, rep)], sem
        ).wait()


def kernel(x, row_w, col_w):
    b = x.shape[0]
    h, w = x.shape[-2], x.shape[-1]
    nc = row_w.shape[1]
    out = pl.pallas_call(
        _pos_kernel,
        in_specs=[
            pl.BlockSpec(memory_space=pltpu.MemorySpace.VMEM),
            pl.BlockSpec(memory_space=pltpu.MemorySpace.VMEM),
        ],
        out_specs=pl.BlockSpec(memory_space=pl.ANY),
        out_shape=jax.ShapeDtypeStruct((b, 2 * nc, h * w), jnp.float32),
        scratch_shapes=[
            pltpu.VMEM((_REP, 2 * nc, h * w), jnp.float32),
            pltpu.SemaphoreType.DMA,
        ],
    )(col_w, row_w)
    return out.reshape(b, 2 * nc, h, w)
